# baseline (speedup 1.0000x reference)
import functools

import jax
import jax.numpy as jnp
from jax.experimental import pallas as pl
from jax.experimental.pallas import tpu as pltpu

_SMOOTH = 0.1
_PAD = 0
_CONF = 1.0 - _SMOOTH

_TR = 32
_TW = 12544
_NT = 8


def _pipeline_kernel(x_hbm, y_ref, out_ref, buf, buft, sems, *, C, n_rb):
    starts = [q * _TW for q in range(_NT - 1)] + [(_NT - 1) * _TW]
    widths = [_TW] * (_NT - 1) + [C - (_NT - 1) * _TW]

    def tile_copy(rb, ct):
        src = x_hbm.at[pl.ds(rb * _TR, _TR), pl.ds(starts[ct], widths[ct])]
        dst = buf.at[ct] if ct < _NT - 1 else buft
        return pltpu.make_async_copy(src, dst, sems.at[ct])

    for ct in range(_NT):
        tile_copy(0, ct).start()

    def body(rb, carry):
        yb = y_ref[pl.ds(rb * _TR, _TR), :]
        ms, ss, ts, gs = [], [], [], []
        x0 = None
        for ct in range(_NT):
            tile_copy(rb, ct).wait()
            xq = buf[ct] if ct < _NT - 1 else buft[...]
            cols = jax.lax.broadcasted_iota(jnp.int32, xq.shape, 1)
            hit = cols == yb - starts[ct]
            mq = jnp.max(xq, axis=1, keepdims=True)
            ms.append(mq)
            ss.append(jnp.sum(jnp.exp(xq - mq), axis=1, keepdims=True))
            ts.append(jnp.sum(xq, axis=1, keepdims=True))
            gs.append(jnp.sum(jnp.where(hit, xq, 0.0), axis=1, keepdims=True))
            if ct == 0:
                x0 = xq[:, 0:1]

            @pl.when(rb + 1 < n_rb)
            def _():
                tile_copy(rb + 1, ct).start()

        m = functools.reduce(jnp.maximum, ms)
        s = sum(sq * jnp.exp(mq - m) for sq, mq in zip(ss, ms))
        t = sum(ts)
        g = sum(gs)

        eps = _SMOOTH / (C - 2)
        K = _SMOOTH * jnp.log(eps) + _CONF * jnp.log(_CONF)
        lse = m + jnp.log(s)
        ssum = t - C * lse
        logp0 = x0 - lse
        logpy = g - lse
        row = K - eps * (ssum - logp0 - logpy) - _CONF * logpy
        out_ref[pl.ds(rb * _TR, _TR), :] = jnp.where(yb != _PAD, row, 0.0)
        return carry

    jax.lax.fori_loop(0, n_rb, body, 0)


@jax.jit
def kernel(x, y):
    B, C = x.shape
    n_rb = B // _TR
    y2 = y.astype(jnp.int32).reshape(B, 1)

    rows = pl.pallas_call(
        functools.partial(_pipeline_kernel, C=C, n_rb=n_rb),
        in_specs=[
            pl.BlockSpec(memory_space=pl.ANY),
            pl.BlockSpec(memory_space=pltpu.MemorySpace.VMEM),
        ],
        out_specs=pl.BlockSpec(memory_space=pltpu.MemorySpace.VMEM),
        out_shape=jax.ShapeDtypeStruct((B, 1), x.dtype),
        scratch_shapes=[
            pltpu.VMEM((_NT - 1, _TR, _TW), jnp.float32),
            pltpu.VMEM((_TR, C - (_NT - 1) * _TW), jnp.float32),
            pltpu.SemaphoreType.DMA((_NT,)),
        ],
    )(x, y2)
    return jnp.sum(rows)
